# Initial kernel scaffold; baseline (speedup 1.0000x reference)
import jax
import jax.numpy as jnp
from jax.experimental import pallas as pl


def kernel(x, edge_index, W1, b1, W2, b2):
    raise NotImplementedError("write your pallas kernel here")


# speedup vs baseline: 6.2919x; 6.2919x over previous
import functools

import jax
import jax.numpy as jnp
from jax import lax
from jax.experimental import pallas as pl
from jax.experimental.pallas import tpu as pltpu
from jax.experimental.pallas import tpu_sc as plsc

N = 10000
E = 160000
C = 256
CH = C // 2
TILES = 16
R_PAD = 10240
ROWS_PER_TILE = R_PAD // TILES
CHUNK = 128
NCHUNKS = 80
EPT = NCHUNKS * CHUNK
E_PAD = EPT * TILES
RB = 512
DEGW = 16

_MESH = plsc.VectorSubcoreMesh(core_axis_name="c", subcore_axis_name="s")


NCH2 = NCHUNKS // 2


@functools.partial(
    pl.kernel,
    mesh=_MESH,
    out_type=jax.ShapeDtypeStruct((2, R_PAD, CHUNK), jnp.float32),
    scratch_types=[
        pltpu.VMEM((NCHUNKS, CHUNK), jnp.int32),
        pltpu.VMEM((CHUNK, CHUNK), jnp.float32),
        pltpu.VMEM_SHARED((R_PAD, CHUNK), jnp.float32),
    ],
)
def _sc_deg(dst_hbm, ones_hbm, zeros_hbm, out_hbm, dst_v, ones_v, acc_sh):
    c = lax.axis_index("c")
    s = lax.axis_index("s")
    rb = s * ROWS_PER_TILE
    pltpu.sync_copy(zeros_hbm, acc_sh.at[pl.ds(rb, ROWS_PER_TILE)])
    pltpu.sync_copy(ones_hbm, ones_v)
    pltpu.sync_copy(dst_hbm.at[s], dst_v)
    plsc.subcore_barrier()
    j0 = c * NCH2

    def body(j, carry):
        pltpu.sync_copy(ones_v, acc_sh.at[dst_v.at[j0 + j]], add=True)
        return carry

    lax.fori_loop(0, NCH2, body, 0)
    plsc.subcore_barrier()
    pltpu.sync_copy(
        acc_sh.at[pl.ds(rb, ROWS_PER_TILE)],
        out_hbm.at[c, pl.ds(rb, ROWS_PER_TILE)],
    )


@functools.partial(
    pl.kernel,
    mesh=_MESH,
    out_type=jax.ShapeDtypeStruct((2, R_PAD, CH), jnp.float32),
    scratch_types=[
        pltpu.VMEM((NCHUNKS, CHUNK), jnp.int32),
        pltpu.VMEM((NCHUNKS, CHUNK), jnp.int32),
        pltpu.VMEM((CHUNK, CH), jnp.float32),
        pltpu.VMEM_SHARED((R_PAD, CH), jnp.float32),
    ],
)
def _sc_msg(g_hbm, src_hbm, dst_hbm, out_hbm, src_v, dst_v, rows_v, acc_sh):
    c = lax.axis_index("c")
    s = lax.axis_index("s")
    rb = s * ROWS_PER_TILE
    pltpu.sync_copy(
        g_hbm.at[c, pl.ds(rb, ROWS_PER_TILE)],
        acc_sh.at[pl.ds(rb, ROWS_PER_TILE)],
    )
    pltpu.sync_copy(src_hbm.at[s], src_v)
    pltpu.sync_copy(dst_hbm.at[s], dst_v)
    plsc.subcore_barrier()

    def body(j, carry):
        pltpu.sync_copy(g_hbm.at[c].at[src_v.at[j]], rows_v)
        pltpu.sync_copy(rows_v, acc_sh.at[dst_v.at[j]], add=True)
        return carry

    lax.fori_loop(0, NCHUNKS, body, 0)
    plsc.subcore_barrier()
    pltpu.sync_copy(
        acc_sh.at[pl.ds(rb, ROWS_PER_TILE)],
        out_hbm.at[c, pl.ds(rb, ROWS_PER_TILE)],
    )


def _tc1_body(x_ref, w_ref, dega_ref, degb_ref, out_ref, deg_ref):
    deg = dega_ref[...] + degb_ref[...]
    deg_ref[...] = deg
    dis = lax.rsqrt(deg + 1.0)
    h = jnp.dot(x_ref[...], w_ref[...],
                preferred_element_type=jnp.float32,
                precision=lax.Precision.HIGHEST)
    out_ref[...] = (h * dis)[None]


_tc1 = pl.pallas_call(
    _tc1_body,
    grid=(R_PAD // RB, 2),
    in_specs=[
        pl.BlockSpec((RB, C), lambda i, j: (i, 0)),
        pl.BlockSpec((C, CH), lambda i, j: (0, j)),
        pl.BlockSpec((RB, 1), lambda i, j: (i, 0)),
        pl.BlockSpec((RB, 1), lambda i, j: (i, 0)),
    ],
    out_specs=[
        pl.BlockSpec((1, RB, CH), lambda i, j: (j, i, 0)),
        pl.BlockSpec((RB, 1), lambda i, j: (i, 0)),
    ],
    out_shape=[
        jax.ShapeDtypeStruct((2, R_PAD, CH), jnp.float32),
        jax.ShapeDtypeStruct((R_PAD, 1), jnp.float32),
    ],
)


def _tc2_body(sa_ref, sb_ref, deg_ref, b_ref, w_ref, out_ref):
    dis = lax.rsqrt(deg_ref[...] + 1.0)
    ha = jnp.maximum(sa_ref[0] * dis + b_ref[0:1, :], 0.0)
    hb = jnp.maximum(sb_ref[0] * dis + b_ref[1:2, :], 0.0)
    w = w_ref[...]
    h = jnp.dot(ha, w[:CH, :], preferred_element_type=jnp.float32,
                precision=lax.Precision.HIGHEST)
    h = h + jnp.dot(hb, w[CH:, :], preferred_element_type=jnp.float32,
                    precision=lax.Precision.HIGHEST)
    out_ref[...] = (h * dis)[None]


_tc2 = pl.pallas_call(
    _tc2_body,
    grid=(R_PAD // RB, 2),
    in_specs=[
        pl.BlockSpec((1, RB, CH), lambda i, j: (0, i, 0)),
        pl.BlockSpec((1, RB, CH), lambda i, j: (1, i, 0)),
        pl.BlockSpec((RB, 1), lambda i, j: (i, 0)),
        pl.BlockSpec((2, CH), lambda i, j: (0, 0)),
        pl.BlockSpec((C, CH), lambda i, j: (0, j)),
    ],
    out_specs=pl.BlockSpec((1, RB, CH), lambda i, j: (j, i, 0)),
    out_shape=jax.ShapeDtypeStruct((2, R_PAD, CH), jnp.float32),
)


def _tc3_body(s_ref, deg_ref, b_ref, out_ref):
    dis = lax.rsqrt(deg_ref[...] + 1.0)
    b = b_ref[...]
    brow = jnp.where(pl.program_id(1) == 0, b[0:1, :], b[1:2, :])
    out_ref[...] = jnp.maximum(s_ref[0] * dis + brow, 0.0)


_tc3 = pl.pallas_call(
    _tc3_body,
    grid=(R_PAD // RB, 2),
    in_specs=[
        pl.BlockSpec((1, RB, CH), lambda i, j: (j, i, 0)),
        pl.BlockSpec((RB, 1), lambda i, j: (i, 0)),
        pl.BlockSpec((2, CH), lambda i, j: (0, 0)),
    ],
    out_specs=pl.BlockSpec((RB, CH), lambda i, j: (i, j)),
    out_shape=jax.ShapeDtypeStruct((R_PAD, C), jnp.float32),
)


def kernel(x, edge_index, W1, b1, W2, b2):
    x_pad = jnp.pad(x, ((0, R_PAD - N), (0, 0)))
    pad_e = E_PAD - E
    src_p = jnp.pad(edge_index[0], (0, pad_e), constant_values=N)
    dst_p = jnp.pad(edge_index[1], (0, pad_e), constant_values=N)
    src_p = src_p.reshape(TILES, NCHUNKS, CHUNK)
    dst_p = dst_p.reshape(TILES, NCHUNKS, CHUNK)
    ones = jnp.ones((CHUNK, CHUNK), jnp.float32)
    zeros = jnp.zeros((ROWS_PER_TILE, CHUNK), jnp.float32)

    degs = _sc_deg(dst_p, ones, zeros)
    g1, deg = _tc1(x_pad, W1, degs[0, :, :1], degs[1, :, :1])
    s1 = _sc_msg(g1, src_p, dst_p)
    g2 = _tc2(s1, s1, deg, b1.reshape(2, CH), W2)
    s2 = _sc_msg(g2, src_p, dst_p)
    out = _tc3(s2, deg, b2.reshape(2, CH))
    return out[:N]
